# baseline (speedup 1.0000x reference)
import functools

import jax
import jax.numpy as jnp
from jax import lax
from jax.experimental import pallas as pl
from jax.experimental.pallas import tpu as pltpu
from jax.experimental.pallas import tpu_sc as plsc


def kernel(U, V, pos_u_idxs, pos_v_idxs, neg_v_idxs):
    B = pos_u_idxs.shape[0]
    K = neg_v_idxs.shape[1]
    D = U.shape[1]
    VOC = U.shape[0]
    L = 16
    RW = 2 * D
    NC, NS = 2, 16
    NW = NC * NS
    BPW = B // NW
    C = 16
    NCHUNK = BPW // C
    KP = 24
    NBUF = 2

    U2 = U.reshape(VOC // 2, RW)
    V2 = V.reshape(VOC // 2, RW)
    neg_flat = neg_v_idxs.reshape(B * K)

    mesh = plsc.VectorSubcoreMesh(
        core_axis_name="c", subcore_axis_name="s",
        num_cores=NC, num_subcores=NS)

    @functools.partial(
        pl.kernel,
        out_type=jax.ShapeDtypeStruct((NW, KP, BPW), jnp.float32),
        mesh=mesh,
        scratch_types=[
            pltpu.VMEM((BPW,), jnp.int32),
            pltpu.VMEM((BPW,), jnp.int32),
            pltpu.VMEM((BPW * K,), jnp.int32),
            pltpu.VMEM((BPW,), jnp.int32),
            pltpu.VMEM((BPW,), jnp.int32),
            pltpu.VMEM((BPW * K,), jnp.int32),
            pltpu.VMEM((NBUF, C, RW), jnp.float32),
            pltpu.VMEM((NBUF, C, RW), jnp.float32),
            pltpu.VMEM((NBUF, C * K, RW), jnp.float32),
            pltpu.VMEM((KP, BPW), jnp.float32),
            pltpu.SemaphoreType.DMA,
            pltpu.SemaphoreType.DMA,
        ],
        compiler_params=pltpu.CompilerParams(
            needs_layout_passes=False, use_tc_tiling_on_sc=True),
    )
    def sc_scores(u_hbm, v_hbm, pu_hbm, pv_hbm, nv_hbm, out_hbm,
                  iu_o, ipv_o, inv_o, iu_h, ipv_h, inv_h,
                  rows_u, rows_pv, rows_nv, scores, sem, sem2):
        wid = lax.axis_index("s") * NC + lax.axis_index("c")
        gbase = wid * BPW
        eye = lax.iota(jnp.int32, L)

        c1 = pltpu.async_copy(pu_hbm.at[pl.ds(gbase, BPW)], iu_o, sem2)
        c2 = pltpu.async_copy(pv_hbm.at[pl.ds(gbase, BPW)], ipv_o, sem2)
        c3 = pltpu.async_copy(nv_hbm.at[pl.ds(gbase * K, BPW * K)], inv_o, sem2)
        c1.wait()
        c2.wait()
        c3.wait()

        def shift_body(g, carry):
            s = g * L
            iu_h[pl.ds(s, L)] = lax.shift_right_logical(iu_o[pl.ds(s, L)], 1)
            ipv_h[pl.ds(s, L)] = lax.shift_right_logical(ipv_o[pl.ds(s, L)], 1)
            for t in range(K):
                sk = g * (K * L) + t * L
                inv_h[pl.ds(sk, L)] = lax.shift_right_logical(
                    inv_o[pl.ds(sk, L)], 1)
            return carry

        lax.fori_loop(0, BPW // L, shift_body, 0)

        def fire(cj):
            lb = cj * C
            bb = lax.rem(cj, NBUF)
            pltpu.async_copy(u_hbm.at[iu_h.at[pl.ds(lb, C)]],
                             rows_u.at[bb], sem)
            pltpu.async_copy(v_hbm.at[ipv_h.at[pl.ds(lb, C)]],
                             rows_pv.at[bb], sem)
            pltpu.async_copy(v_hbm.at[inv_h.at[pl.ds(lb * K, C * K)]],
                             rows_nv.at[bb], sem)

        def drain(cj):
            lb = cj * C
            bb = lax.rem(cj, NBUF)
            pltpu.make_async_copy(u_hbm.at[iu_h.at[pl.ds(lb, C)]],
                                  rows_u.at[bb], sem).wait()
            pltpu.make_async_copy(v_hbm.at[ipv_h.at[pl.ds(lb, C)]],
                                  rows_pv.at[bb], sem).wait()
            pltpu.make_async_copy(v_hbm.at[inv_h.at[pl.ds(lb * K, C * K)]],
                                  rows_nv.at[bb], sem).wait()

        fire(0)

        def chunk_body(ci, carry):
            lb = ci * C
            bb = lax.rem(ci, NBUF)
            drain(ci)

            @pl.when(ci + 1 < NCHUNK)
            def _():
                fire(ci + 1)

            bbv = jnp.broadcast_to(bb, (L,)).astype(jnp.int32)
            offu = lax.shift_left(iu_o[pl.ds(lb, L)] & 1, 6)
            offpv = lax.shift_left(ipv_o[pl.ds(lb, L)] & 1, 6)
            offn = [
                lax.shift_left(
                    plsc.load_gather(inv_o, [eye * K + (lb * K + t)]) & 1, 6)
                for t in range(K)
            ]
            nrow = [eye * K + t for t in range(K)]

            def make_pass(klist, with_pos):
                def dbody(d, accs):
                    ud = plsc.load_gather(rows_u, [bbv, eye, offu + d])
                    new = []
                    i = 0
                    if with_pos:
                        pvd = plsc.load_gather(rows_pv, [bbv, eye, offpv + d])
                        new.append(accs[0] + ud * pvd)
                        i = 1
                    for t in klist:
                        nvd = plsc.load_gather(
                            rows_nv, [bbv, nrow[t], offn[t] + d])
                        new.append(accs[i] + ud * nvd)
                        i += 1
                    return tuple(new)
                n = len(klist) + (1 if with_pos else 0)
                zero = jnp.zeros((L,), jnp.float32)
                return lax.fori_loop(0, D, dbody, (zero,) * n, unroll=8)

            r1 = make_pass(list(range(0, 6)), with_pos=True)
            r2 = make_pass(list(range(6, 13)), with_pos=False)
            r3 = make_pass(list(range(13, 20)), with_pos=False)

            scores[0, pl.ds(lb, L)] = r1[0]
            for t in range(6):
                scores[1 + t, pl.ds(lb, L)] = -r1[1 + t]
            for t in range(6, 13):
                scores[1 + t, pl.ds(lb, L)] = -r2[t - 6]
            for t in range(13, 20):
                scores[1 + t, pl.ds(lb, L)] = -r3[t - 13]
            return carry

        lax.fori_loop(0, NCHUNK, chunk_body, 0)
        pltpu.sync_copy(scores, out_hbm.at[wid])

    scores3 = sc_scores(U2, V2, pos_u_idxs, pos_v_idxs, neg_flat)

    def tc_body(s_ref, o_ref):
        x = s_ref[...]
        krow = lax.broadcasted_iota(jnp.int32, x.shape, 1)
        valid = krow < (1 + K)
        ls = jnp.where(valid, jnp.log(jax.nn.sigmoid(x)), 0.0)
        o_ref[0, 0] = -jnp.sum(ls) / B

    loss = pl.pallas_call(
        tc_body,
        out_shape=jax.ShapeDtypeStruct((1, 1), jnp.float32),
        out_specs=pl.BlockSpec(memory_space=pltpu.SMEM),
    )(scores3)
    return loss[0, 0]


# speedup vs baseline: 1.1551x; 1.1551x over previous
import functools

import jax
import jax.numpy as jnp
from jax import lax
from jax.experimental import pallas as pl
from jax.experimental.pallas import tpu as pltpu
from jax.experimental.pallas import tpu_sc as plsc


def kernel(U, V, pos_u_idxs, pos_v_idxs, neg_v_idxs):
    B = pos_u_idxs.shape[0]
    K = neg_v_idxs.shape[1]
    D = U.shape[1]
    L = 16
    RW = 2 * D
    NC, NS = 2, 16
    NW = NC * NS
    BPW = B // NW
    C = 16
    NCHUNK = BPW // C
    KP = 24
    NBUF = 2

    W = jnp.concatenate([U, V], axis=1)
    neg_flat = neg_v_idxs.reshape(B * K)

    mesh = plsc.VectorSubcoreMesh(
        core_axis_name="c", subcore_axis_name="s",
        num_cores=NC, num_subcores=NS)

    @functools.partial(
        pl.kernel,
        out_type=jax.ShapeDtypeStruct((NW, KP, BPW), jnp.float32),
        mesh=mesh,
        scratch_types=[
            pltpu.VMEM((BPW,), jnp.int32),
            pltpu.VMEM((BPW,), jnp.int32),
            pltpu.VMEM((BPW * K,), jnp.int32),
            pltpu.VMEM((NBUF, C, RW), jnp.float32),
            pltpu.VMEM((NBUF, C, RW), jnp.float32),
            pltpu.VMEM((NBUF, C * K, RW), jnp.float32),
            pltpu.VMEM((KP, BPW), jnp.float32),
            pltpu.SemaphoreType.DMA,
            pltpu.SemaphoreType.DMA,
        ],
        compiler_params=pltpu.CompilerParams(
            needs_layout_passes=False, use_tc_tiling_on_sc=True),
    )
    def sc_scores(w_hbm, pu_hbm, pv_hbm, nv_hbm, out_hbm,
                  iu, ipv, inv, rows_u, rows_pv, rows_nv, scores, sem, sem2):
        wid = lax.axis_index("s") * NC + lax.axis_index("c")
        gbase = wid * BPW
        eye = lax.iota(jnp.int32, L)

        c1 = pltpu.async_copy(pu_hbm.at[pl.ds(gbase, BPW)], iu, sem2)
        c2 = pltpu.async_copy(pv_hbm.at[pl.ds(gbase, BPW)], ipv, sem2)
        c3 = pltpu.async_copy(nv_hbm.at[pl.ds(gbase * K, BPW * K)], inv, sem2)
        c1.wait()
        c2.wait()
        c3.wait()

        def fire(cj):
            lb = cj * C
            bb = lax.rem(cj, NBUF)
            pltpu.async_copy(w_hbm.at[iu.at[pl.ds(lb, C)]],
                             rows_u.at[bb], sem)
            pltpu.async_copy(w_hbm.at[ipv.at[pl.ds(lb, C)]],
                             rows_pv.at[bb], sem)
            pltpu.async_copy(w_hbm.at[inv.at[pl.ds(lb * K, C * K)]],
                             rows_nv.at[bb], sem)

        def drain(cj):
            lb = cj * C
            bb = lax.rem(cj, NBUF)
            pltpu.make_async_copy(w_hbm.at[iu.at[pl.ds(lb, C)]],
                                  rows_u.at[bb], sem).wait()
            pltpu.make_async_copy(w_hbm.at[ipv.at[pl.ds(lb, C)]],
                                  rows_pv.at[bb], sem).wait()
            pltpu.make_async_copy(w_hbm.at[inv.at[pl.ds(lb * K, C * K)]],
                                  rows_nv.at[bb], sem).wait()

        fire(0)

        def chunk_body(ci, carry):
            lb = ci * C
            bb = lax.rem(ci, NBUF)
            drain(ci)

            @pl.when(ci + 1 < NCHUNK)
            def _():
                fire(ci + 1)

            bbv = jnp.broadcast_to(bb, (L,)).astype(jnp.int32)
            nrow = [eye * K + t for t in range(K)]

            def make_pass(klist, with_pos):
                def dbody(d, accs):
                    du = jnp.broadcast_to(d, (L,)).astype(jnp.int32)
                    dv = du + D
                    ud = plsc.load_gather(rows_u, [bbv, eye, du])
                    new = []
                    i = 0
                    if with_pos:
                        pvd = plsc.load_gather(rows_pv, [bbv, eye, dv])
                        new.append(accs[0] + ud * pvd)
                        i = 1
                    for t in klist:
                        nvd = plsc.load_gather(rows_nv, [bbv, nrow[t], dv])
                        new.append(accs[i] + ud * nvd)
                        i += 1
                    return tuple(new)
                n = len(klist) + (1 if with_pos else 0)
                zero = jnp.zeros((L,), jnp.float32)
                return lax.fori_loop(0, D, dbody, (zero,) * n, unroll=8)

            r1 = make_pass(list(range(0, 6)), with_pos=True)
            r2 = make_pass(list(range(6, 13)), with_pos=False)
            r3 = make_pass(list(range(13, 20)), with_pos=False)

            scores[0, pl.ds(lb, L)] = r1[0]
            for t in range(6):
                scores[1 + t, pl.ds(lb, L)] = -r1[1 + t]
            for t in range(6, 13):
                scores[1 + t, pl.ds(lb, L)] = -r2[t - 6]
            for t in range(13, 20):
                scores[1 + t, pl.ds(lb, L)] = -r3[t - 13]
            return carry

        lax.fori_loop(0, NCHUNK, chunk_body, 0)
        pltpu.sync_copy(scores, out_hbm.at[wid])

    scores3 = sc_scores(W, pos_u_idxs, pos_v_idxs, neg_flat)

    def tc_body(s_ref, o_ref):
        x = s_ref[...]
        krow = lax.broadcasted_iota(jnp.int32, x.shape, 1)
        valid = krow < (1 + K)
        ls = jnp.where(valid, jnp.log(jax.nn.sigmoid(x)), 0.0)
        o_ref[0, 0] = -jnp.sum(ls) / B

    loss = pl.pallas_call(
        tc_body,
        out_shape=jax.ShapeDtypeStruct((1, 1), jnp.float32),
        out_specs=pl.BlockSpec(memory_space=pltpu.SMEM),
    )(scores3)
    return loss[0, 0]
